# Initial kernel scaffold; baseline (speedup 1.0000x reference)
import jax
import jax.numpy as jnp
from jax.experimental import pallas as pl


def kernel(x, y, z, outputPos, posEmbeddingList):
    raise NotImplementedError("write your pallas kernel here")


# speedup vs baseline: 2.2188x; 2.2188x over previous
import functools

import jax
import jax.numpy as jnp
from jax import lax
from jax.experimental import pallas as pl
from jax.experimental.pallas import tpu as pltpu
from jax.experimental.pallas import tpu_sc as plsc

RES = 16
D = 64
B = 32
N = 8192
TABLE = RES ** 3

NC = 2
NS = 16
NW = NC * NS

CHUNK = 512
NCHUNK = N // CHUNK
LANES = 16
MAGIC = 12582912.0


def _sc_body(x_hbm, y_hbm, z_hbm, op_hbm, table_hbm, out_hbm,
             xv, yv, zv, idx_s, buf, rowbuf, opv, gsem):
    w = lax.axis_index("s") * NC + lax.axis_index("c")
    base = pl.multiple_of(w * N, N)

    pltpu.sync_copy(x_hbm.at[pl.ds(base, N)], xv)
    pltpu.sync_copy(y_hbm.at[pl.ds(base, N)], yv)
    pltpu.sync_copy(z_hbm.at[pl.ds(base, N)], zv)
    pltpu.sync_copy(op_hbm, opv)

    idx_s[pl.ds(0, LANES)] = jnp.zeros((LANES,), jnp.int32)

    def step(i, carry):
        s = pl.ds(i * LANES, LANES)
        rx = (xv[s] * 15.0 + MAGIC) - MAGIC
        ry = (yv[s] * 15.0 + MAGIC) - MAGIC
        rz = (zv[s] * 15.0 + MAGIC) - MAGIC
        pos = rx * 256.0 + ry * 16.0 + rz
        idx_s[pl.ds(i * LANES + 1, LANES)] = pos.astype(jnp.int32)
        return carry

    lax.fori_loop(0, N // LANES, step, 0)

    def chunk_step(c, carry):
        off = pl.multiple_of(c * CHUNK, CHUNK)
        idx_slice = idx_s.at[pl.ds(off, CHUNK)]
        pltpu.async_copy(table_hbm.at[idx_slice], buf, gsem).wait()
        pltpu.sync_copy(buf, out_hbm.at[w, pl.ds(off, CHUNK)])
        return carry

    lax.fori_loop(0, NCHUNK, chunk_step, 0)

    pltpu.async_copy(table_hbm.at[idx_s.at[pl.ds(N, 1)]], rowbuf, gsem).wait()
    pltpu.sync_copy(rowbuf, out_hbm.at[w, pl.ds(N, 1)])
    pltpu.sync_copy(opv, out_hbm.at[w, pl.ds(0, 1)])


@functools.partial(
    pl.kernel,
    mesh=plsc.VectorSubcoreMesh(core_axis_name="c", subcore_axis_name="s"),
    compiler_params=pltpu.CompilerParams(use_tc_tiling_on_sc=False),
    out_type=jax.ShapeDtypeStruct((B, N + 1, D), jnp.float32),
    scratch_types=[
        pltpu.VMEM((N,), jnp.float32),
        pltpu.VMEM((N,), jnp.float32),
        pltpu.VMEM((N,), jnp.float32),
        pltpu.VMEM((N + LANES,), jnp.int32),
        pltpu.VMEM((CHUNK, D), jnp.float32),
        pltpu.VMEM((1, D), jnp.float32),
        pltpu.VMEM((1, D), jnp.float32),
        pltpu.SemaphoreType.DMA,
    ],
)
def _sc_kernel(x_hbm, y_hbm, z_hbm, op_hbm, table_hbm, out_hbm,
               xv, yv, zv, idx_s, buf, rowbuf, opv, gsem):
    _sc_body(x_hbm, y_hbm, z_hbm, op_hbm, table_hbm, out_hbm,
             xv, yv, zv, idx_s, buf, rowbuf, opv, gsem)


def kernel(x, y, z, outputPos, posEmbeddingList):
    return _sc_kernel(x.reshape(-1), y.reshape(-1), z.reshape(-1),
                      outputPos, posEmbeddingList)


# speedup vs baseline: 2.3960x; 1.0799x over previous
import functools

import jax
import jax.numpy as jnp
from jax import lax
from jax.experimental import pallas as pl
from jax.experimental.pallas import tpu as pltpu
from jax.experimental.pallas import tpu_sc as plsc

RES = 16
D = 64
B = 32
N = 8192
TABLE = RES ** 3

NC = 2
NS = 16
NW = NC * NS

CHUNK = 512
NCHUNK = N // CHUNK
LANES = 16
MAGIC = 12582912.0


def _sc_body(x_hbm, y_hbm, z_hbm, op_hbm, table_hbm, out_hbm,
             table_sh, xv, yv, zv, idx_s, buf0, buf1, rowbuf, opv,
             gsem0, gsem1, ssem):
    s = lax.axis_index("s")
    w = s * NC + lax.axis_index("c")
    base = pl.multiple_of(w * N, N)

    @pl.when(s == 0)
    def _():
        pltpu.sync_copy(table_hbm, table_sh)

    pltpu.async_copy(x_hbm.at[pl.ds(base, N)], xv, ssem).wait()
    pltpu.async_copy(y_hbm.at[pl.ds(base, N)], yv, ssem).wait()
    pltpu.async_copy(z_hbm.at[pl.ds(base, N)], zv, ssem).wait()
    pltpu.async_copy(op_hbm, opv, ssem).wait()

    idx_s[pl.ds(0, LANES)] = jnp.zeros((LANES,), jnp.int32)

    def step(i, carry):
        sl = pl.ds(i * LANES, LANES)
        rx = (xv[sl] * 15.0 + MAGIC) - MAGIC
        ry = (yv[sl] * 15.0 + MAGIC) - MAGIC
        rz = (zv[sl] * 15.0 + MAGIC) - MAGIC
        pos = rx * 256.0 + ry * 16.0 + rz
        idx_s[pl.ds(i * LANES + 1, LANES)] = pos.astype(jnp.int32)
        return carry

    lax.fori_loop(0, N // LANES, step, 0)

    plsc.subcore_barrier()

    bufs = (buf0, buf1)
    sems = (gsem0, gsem1)

    def gather(c, b):
        off = pl.multiple_of(c * CHUNK, CHUNK)
        return pltpu.async_copy(table_sh.at[idx_s.at[pl.ds(off, CHUNK)]],
                                bufs[b], sems[b])

    def write(c, b):
        off = pl.multiple_of(c * CHUNK, CHUNK)
        pltpu.sync_copy(bufs[b], out_hbm.at[w, pl.ds(off, CHUNK)])

    pending = gather(0, 0)
    for c in range(NCHUNK):
        b = c % 2
        pending.wait()
        if c + 1 < NCHUNK:
            pending = gather(c + 1, (c + 1) % 2)
        write(c, b)

    pltpu.async_copy(table_sh.at[idx_s.at[pl.ds(N, 1)]], rowbuf, gsem0).wait()
    pltpu.sync_copy(rowbuf, out_hbm.at[w, pl.ds(N, 1)])
    pltpu.sync_copy(opv, out_hbm.at[w, pl.ds(0, 1)])


@functools.partial(
    pl.kernel,
    mesh=plsc.VectorSubcoreMesh(core_axis_name="c", subcore_axis_name="s"),
    compiler_params=pltpu.CompilerParams(use_tc_tiling_on_sc=False),
    out_type=jax.ShapeDtypeStruct((B, N + 1, D), jnp.float32),
    scratch_types=[
        pltpu.VMEM_SHARED((TABLE, D), jnp.float32),
        pltpu.VMEM((N,), jnp.float32),
        pltpu.VMEM((N,), jnp.float32),
        pltpu.VMEM((N,), jnp.float32),
        pltpu.VMEM((N + LANES,), jnp.int32),
        pltpu.VMEM((CHUNK, D), jnp.float32),
        pltpu.VMEM((CHUNK, D), jnp.float32),
        pltpu.VMEM((1, D), jnp.float32),
        pltpu.VMEM((1, D), jnp.float32),
        pltpu.SemaphoreType.DMA,
        pltpu.SemaphoreType.DMA,
        pltpu.SemaphoreType.DMA,
    ],
)
def _sc_kernel(x_hbm, y_hbm, z_hbm, op_hbm, table_hbm, out_hbm,
               table_sh, xv, yv, zv, idx_s, buf0, buf1, rowbuf, opv,
               gsem0, gsem1, ssem):
    _sc_body(x_hbm, y_hbm, z_hbm, op_hbm, table_hbm, out_hbm,
             table_sh, xv, yv, zv, idx_s, buf0, buf1, rowbuf, opv,
             gsem0, gsem1, ssem)


def kernel(x, y, z, outputPos, posEmbeddingList):
    return _sc_kernel(x.reshape(-1), y.reshape(-1), z.reshape(-1),
                      outputPos, posEmbeddingList)


# speedup vs baseline: 4.8221x; 2.0126x over previous
import functools

import jax
import jax.numpy as jnp
from jax import lax
from jax.experimental import pallas as pl
from jax.experimental.pallas import tpu as pltpu
from jax.experimental.pallas import tpu_sc as plsc

RES = 16
D = 64
B = 32
N = 8192
TABLE = RES ** 3

NC = 2
NS = 16
DG = 8
NB_SC = 16

LANES = 16
MAGIC = 12582912.0
CW = 2048
SLOT = N + 128
HA = 4096
WB = N + 1 - HA


def _sc_body(x_hbm, y_hbm, z_hbm, op_hbm, t_hbm, out_hbm,
             idx_all, idxb, xv, yv, zv, tvm, bufA, bufB, opvm,
             sem_tab, sem_wA, sem_wB):
    c = lax.axis_index("c")
    s = lax.axis_index("s")
    b1 = c * NB_SC + s
    base = pl.multiple_of(b1 * N, N)
    g = s % DG
    sg = s // DG
    d0 = pl.multiple_of(g * DG, DG)

    iota = lax.iota(jnp.int32, LANES)

    tab_cp = pltpu.async_copy(
        t_hbm.at[pl.ds(pl.multiple_of(g * (DG * TABLE), 128), DG * TABLE)],
        tvm, sem_tab)
    pltpu.sync_copy(op_hbm, opvm)

    idxb[pl.ds(0, LANES)] = jnp.zeros((LANES,), jnp.int32)

    for ch in range(N // CW):
        coff = pl.multiple_of(base + ch * CW, CW)
        pltpu.sync_copy(x_hbm.at[pl.ds(coff, CW)], xv)
        pltpu.sync_copy(y_hbm.at[pl.ds(coff, CW)], yv)
        pltpu.sync_copy(z_hbm.at[pl.ds(coff, CW)], zv)

        def step(i, carry, _ch=ch):
            sl = pl.ds(i * LANES, LANES)
            rx = (xv[sl] * 15.0 + MAGIC) - MAGIC
            ry = (yv[sl] * 15.0 + MAGIC) - MAGIC
            rz = (zv[sl] * 15.0 + MAGIC) - MAGIC
            pos = (rx * 256.0 + ry * 16.0 + rz).astype(jnp.int32)
            plsc.store_scatter(idxb, [iota + (i * LANES + (_ch * CW + 1))], pos)
            return carry

        lax.fori_loop(0, CW // LANES, step, 0)

    slt = pl.ds(CW - LANES, LANES)
    rx = (xv[slt] * 15.0 + MAGIC) - MAGIC
    ry = (yv[slt] * 15.0 + MAGIC) - MAGIC
    rz = (zv[slt] * 15.0 + MAGIC) - MAGIC
    pos_t = (rx * 256.0 + ry * 16.0 + rz).astype(jnp.int32)
    plsc.store_scatter(idxb, [(N + 15) - iota], pos_t)

    pltpu.sync_copy(idxb, idx_all.at[pl.ds(pl.multiple_of(s * SLOT, 16), SLOT)])
    tab_cp.wait()
    plsc.subcore_barrier()

    pend_A = pend_B = None
    for k in range(DG):
        bl = sg * DG + k
        b = c * NB_SC + bl
        pltpu.sync_copy(idx_all.at[pl.ds(pl.multiple_of(bl * SLOT, 16), SLOT)],
                        idxb)

        if pend_A is not None:
            pend_A.wait()

        def stepA(j, carry):
            off = pl.multiple_of(j * LANES, LANES)
            iv = idxb[pl.ds(off, LANES)]
            for dd in range(DG):
                gv = plsc.load_gather(tvm, [iv + (dd * TABLE)])
                bufA[dd, pl.ds(off, LANES)] = gv
            return carry

        lax.fori_loop(0, HA // LANES, stepA, 0)

        opvec = opvm[pl.ds(d0, LANES)]
        plsc.store_scatter(bufA, [iota, jnp.zeros((LANES,), jnp.int32)],
                           opvec, mask=iota < DG)
        pend_A = pltpu.async_copy(
            bufA, out_hbm.at[b, pl.ds(d0, DG), pl.ds(0, HA)], sem_wA)

        if pend_B is not None:
            pend_B.wait()

        def stepB(j, carry):
            off = pl.multiple_of(j * LANES, LANES)
            iv = idxb[pl.ds(HA + off, LANES)]
            for dd in range(DG):
                gv = plsc.load_gather(tvm, [iv + (dd * TABLE)])
                bufB[dd, pl.ds(off, LANES)] = gv
            return carry

        lax.fori_loop(0, (WB - 1) // LANES, stepB, 0)

        ivt = idxb[pl.ds(N, LANES)]
        for dd in range(DG):
            gv = plsc.load_gather(tvm, [ivt + (dd * TABLE)])
            plsc.store_scatter(bufB, [jnp.full((LANES,), dd, jnp.int32),
                                      jnp.full((LANES,), WB - 1, jnp.int32)],
                               gv, mask=iota < 1)
        pend_B = pltpu.async_copy(
            bufB, out_hbm.at[b, pl.ds(d0, DG), pl.ds(HA, WB)], sem_wB)

    pend_A.wait()
    pend_B.wait()


@functools.partial(
    pl.kernel,
    mesh=plsc.VectorSubcoreMesh(core_axis_name="c", subcore_axis_name="s"),
    compiler_params=pltpu.CompilerParams(use_tc_tiling_on_sc=True,
                                         needs_layout_passes=False),
    out_type=jax.ShapeDtypeStruct((B, D, N + 1), jnp.float32),
    scratch_types=[
        pltpu.VMEM_SHARED((NB_SC * SLOT,), jnp.int32),
        pltpu.VMEM((SLOT,), jnp.int32),
        pltpu.VMEM((CW,), jnp.float32),
        pltpu.VMEM((CW,), jnp.float32),
        pltpu.VMEM((CW,), jnp.float32),
        pltpu.VMEM((DG * TABLE,), jnp.float32),
        pltpu.VMEM((DG, HA), jnp.float32),
        pltpu.VMEM((DG, WB), jnp.float32),
        pltpu.VMEM((D + LANES,), jnp.float32),
        pltpu.SemaphoreType.DMA,
        pltpu.SemaphoreType.DMA,
        pltpu.SemaphoreType.DMA,
    ],
)
def _sc_kernel(x_hbm, y_hbm, z_hbm, op_hbm, t_hbm, out_hbm,
               idx_all, idxb, xv, yv, zv, tvm, bufA, bufB, opvm,
               sem_tab, sem_wA, sem_wB):
    _sc_body(x_hbm, y_hbm, z_hbm, op_hbm, t_hbm, out_hbm,
             idx_all, idxb, xv, yv, zv, tvm, bufA, bufB, opvm,
             sem_tab, sem_wA, sem_wB)


def kernel(x, y, z, outputPos, posEmbeddingList):
    tflat = posEmbeddingList.T.reshape(-1)
    op_pad = jnp.pad(outputPos.reshape(-1), (0, LANES))
    out_t = _sc_kernel(x.reshape(-1), y.reshape(-1), z.reshape(-1),
                       op_pad, tflat)
    return out_t.transpose(0, 2, 1)


# speedup vs baseline: 13.0750x; 2.7115x over previous
import functools

import jax
import jax.numpy as jnp
from jax import lax
from jax.experimental import pallas as pl
from jax.experimental.pallas import tpu as pltpu
from jax.experimental.pallas import tpu_sc as plsc

RES = 16
D = 64
B = 32
N = 8192
TABLE = RES ** 3

NC = 2
NS = 16
DG = 8
NB_SC = 16

LANES = 16
MAGIC = 12582912.0
CW = 2048
SLOT = N + 128
HA = 4096
WB = N + 1 - HA


def _sc_body(x_hbm, y_hbm, z_hbm, op_hbm, t_hbm, out_hbm,
             idx_all, idxb, xv, yv, zv, tvm, bufA, bufB, opvm,
             sem_tab, sem_wA, sem_wB):
    c = lax.axis_index("c")
    s = lax.axis_index("s")
    b1 = c * NB_SC + s
    base = pl.multiple_of(b1 * N, N)
    g = s % DG
    sg = s // DG
    d0 = pl.multiple_of(g * DG, DG)

    iota = lax.iota(jnp.int32, LANES)

    tab_cp = pltpu.async_copy(
        t_hbm.at[pl.ds(pl.multiple_of(g * (DG * TABLE), 128), DG * TABLE)],
        tvm, sem_tab)
    pltpu.sync_copy(op_hbm, opvm)

    idxb[pl.ds(0, LANES)] = jnp.zeros((LANES,), jnp.int32)

    for ch in range(N // CW):
        coff = pl.multiple_of(base + ch * CW, CW)
        pltpu.sync_copy(x_hbm.at[pl.ds(coff, CW)], xv)
        pltpu.sync_copy(y_hbm.at[pl.ds(coff, CW)], yv)
        pltpu.sync_copy(z_hbm.at[pl.ds(coff, CW)], zv)

        @plsc.parallel_loop(0, CW // LANES, unroll=4)
        def _step(i, _ch=ch):
            sl = pl.ds(i * LANES, LANES)
            rx = (xv[sl] * 15.0 + MAGIC) - MAGIC
            ry = (yv[sl] * 15.0 + MAGIC) - MAGIC
            rz = (zv[sl] * 15.0 + MAGIC) - MAGIC
            pos = (rx * 256.0 + ry * 16.0 + rz).astype(jnp.int32)
            plsc.store_scatter(idxb, [iota + (i * LANES + (_ch * CW + 1))], pos)

    slt = pl.ds(CW - LANES, LANES)
    rx = (xv[slt] * 15.0 + MAGIC) - MAGIC
    ry = (yv[slt] * 15.0 + MAGIC) - MAGIC
    rz = (zv[slt] * 15.0 + MAGIC) - MAGIC
    pos_t = (rx * 256.0 + ry * 16.0 + rz).astype(jnp.int32)
    plsc.store_scatter(idxb, [(N + 15) - iota], pos_t)

    pltpu.sync_copy(idxb, idx_all.at[pl.ds(pl.multiple_of(s * SLOT, 16), SLOT)])
    tab_cp.wait()
    plsc.subcore_barrier()

    pend_A = pend_B = None
    for k in range(DG):
        bl = sg * DG + k
        b = c * NB_SC + bl
        pltpu.sync_copy(idx_all.at[pl.ds(pl.multiple_of(bl * SLOT, 16), SLOT)],
                        idxb)

        if pend_A is not None:
            pend_A.wait()

        @plsc.parallel_loop(0, HA // LANES, unroll=4)
        def _stepA(j):
            off = pl.multiple_of(j * LANES, LANES)
            iv = idxb[pl.ds(off, LANES)]
            for dd in range(DG):
                gv = plsc.load_gather(tvm, [iv + (dd * TABLE)])
                bufA[dd, pl.ds(off, LANES)] = gv

        opvec = opvm[pl.ds(d0, LANES)]
        plsc.store_scatter(bufA, [iota, jnp.zeros((LANES,), jnp.int32)],
                           opvec, mask=iota < DG)
        pend_A = pltpu.async_copy(
            bufA, out_hbm.at[b, pl.ds(d0, DG), pl.ds(0, HA)], sem_wA)

        if pend_B is not None:
            pend_B.wait()

        @plsc.parallel_loop(0, (WB - 1) // LANES, unroll=4)
        def _stepB(j):
            off = pl.multiple_of(j * LANES, LANES)
            iv = idxb[pl.ds(HA + off, LANES)]
            for dd in range(DG):
                gv = plsc.load_gather(tvm, [iv + (dd * TABLE)])
                bufB[dd, pl.ds(off, LANES)] = gv

        ivt = idxb[pl.ds(N, LANES)]
        for dd in range(DG):
            gv = plsc.load_gather(tvm, [ivt + (dd * TABLE)])
            plsc.store_scatter(bufB, [jnp.full((LANES,), dd, jnp.int32),
                                      jnp.full((LANES,), WB - 1, jnp.int32)],
                               gv, mask=iota < 1)
        pend_B = pltpu.async_copy(
            bufB, out_hbm.at[b, pl.ds(d0, DG), pl.ds(HA, WB)], sem_wB)

    pend_A.wait()
    pend_B.wait()


@functools.partial(
    pl.kernel,
    mesh=plsc.VectorSubcoreMesh(core_axis_name="c", subcore_axis_name="s"),
    compiler_params=pltpu.CompilerParams(use_tc_tiling_on_sc=True,
                                         needs_layout_passes=False),
    out_type=jax.ShapeDtypeStruct((B, D, N + 1), jnp.float32),
    scratch_types=[
        pltpu.VMEM_SHARED((NB_SC * SLOT,), jnp.int32),
        pltpu.VMEM((SLOT,), jnp.int32),
        pltpu.VMEM((CW,), jnp.float32),
        pltpu.VMEM((CW,), jnp.float32),
        pltpu.VMEM((CW,), jnp.float32),
        pltpu.VMEM((DG * TABLE,), jnp.float32),
        pltpu.VMEM((DG, HA), jnp.float32),
        pltpu.VMEM((DG, WB), jnp.float32),
        pltpu.VMEM((D + LANES,), jnp.float32),
        pltpu.SemaphoreType.DMA,
        pltpu.SemaphoreType.DMA,
        pltpu.SemaphoreType.DMA,
    ],
)
def _sc_kernel(x_hbm, y_hbm, z_hbm, op_hbm, t_hbm, out_hbm,
               idx_all, idxb, xv, yv, zv, tvm, bufA, bufB, opvm,
               sem_tab, sem_wA, sem_wB):
    _sc_body(x_hbm, y_hbm, z_hbm, op_hbm, t_hbm, out_hbm,
             idx_all, idxb, xv, yv, zv, tvm, bufA, bufB, opvm,
             sem_tab, sem_wA, sem_wB)


def kernel(x, y, z, outputPos, posEmbeddingList):
    tflat = posEmbeddingList.T.reshape(-1)
    op_pad = jnp.pad(outputPos.reshape(-1), (0, LANES))
    out_t = _sc_kernel(x.reshape(-1), y.reshape(-1), z.reshape(-1),
                       op_pad, tflat)
    return out_t.transpose(0, 2, 1)


# speedup vs baseline: 13.1166x; 1.0032x over previous
import functools

import jax
import jax.numpy as jnp
from jax import lax
from jax.experimental import pallas as pl
from jax.experimental.pallas import tpu as pltpu
from jax.experimental.pallas import tpu_sc as plsc

RES = 16
D = 64
B = 32
N = 8192
TABLE = RES ** 3

NC = 2
NS = 16
DG = 8
NB_SC = 16

LANES = 16
MAGIC = 12582912.0
CW = 2048
SLOT = N + 128
HA = 4096
WB = N + 1 - HA


def _sc_body(x_hbm, y_hbm, z_hbm, op_hbm, t_hbm, out_hbm,
             idx_all, idxb, xv, yv, zv, tvm, bufA, bufB, opvm,
             sem_tab, sem_wA, sem_wB, sem_pub):
    c = lax.axis_index("c")
    s = lax.axis_index("s")
    b1 = c * NB_SC + s
    base = pl.multiple_of(b1 * N, N)
    g = s % DG
    sg = s // DG
    d0 = pl.multiple_of(g * DG, DG)

    iota = lax.iota(jnp.int32, LANES)

    tab_cp = pltpu.async_copy(
        t_hbm.at[pl.ds(pl.multiple_of(g * (DG * TABLE), 128), DG * TABLE)],
        tvm, sem_tab)
    pltpu.sync_copy(op_hbm, opvm)

    idxb[pl.ds(0, LANES)] = jnp.zeros((LANES,), jnp.int32)

    for ch in range(N // CW):
        coff = pl.multiple_of(base + ch * CW, CW)
        pltpu.sync_copy(x_hbm.at[pl.ds(coff, CW)], xv)
        pltpu.sync_copy(y_hbm.at[pl.ds(coff, CW)], yv)
        pltpu.sync_copy(z_hbm.at[pl.ds(coff, CW)], zv)

        @plsc.parallel_loop(0, CW // LANES, unroll=4)
        def _step(i, _ch=ch):
            sl = pl.ds(i * LANES, LANES)
            rx = (xv[sl] * 15.0 + MAGIC) - MAGIC
            ry = (yv[sl] * 15.0 + MAGIC) - MAGIC
            rz = (zv[sl] * 15.0 + MAGIC) - MAGIC
            pos = (rx * 256.0 + ry * 16.0 + rz).astype(jnp.int32)
            plsc.store_scatter(idxb, [iota + (i * LANES + (_ch * CW + 1))], pos)

    slt = pl.ds(CW - LANES, LANES)
    rx = (xv[slt] * 15.0 + MAGIC) - MAGIC
    ry = (yv[slt] * 15.0 + MAGIC) - MAGIC
    rz = (zv[slt] * 15.0 + MAGIC) - MAGIC
    pos_t = (rx * 256.0 + ry * 16.0 + rz).astype(jnp.int32)
    plsc.store_scatter(idxb, [(N + 15) - iota], pos_t)

    pub_cp = pltpu.async_copy(
        idxb, idx_all.at[pl.ds(pl.multiple_of(s * SLOT, 16), SLOT)], sem_pub)
    tab_cp.wait()

    pend_A = pend_B = None
    for ki in range(DG):
        k = (g + ki) % DG
        bl = sg * DG + k
        b = c * NB_SC + bl
        if ki == 1:
            pub_cp.wait()
            plsc.subcore_barrier()
        if ki > 0:
            pltpu.sync_copy(
                idx_all.at[pl.ds(pl.multiple_of(bl * SLOT, 16), SLOT)], idxb)

        if pend_A is not None:
            pend_A.wait()

        @plsc.parallel_loop(0, HA // LANES, unroll=8)
        def _stepA(j):
            off = pl.multiple_of(j * LANES, LANES)
            iv = idxb[pl.ds(off, LANES)]
            for dd in range(DG):
                gv = plsc.load_gather(tvm, [iv + (dd * TABLE)])
                bufA[dd, pl.ds(off, LANES)] = gv

        opvec = opvm[pl.ds(d0, LANES)]
        plsc.store_scatter(bufA, [iota, jnp.zeros((LANES,), jnp.int32)],
                           opvec, mask=iota < DG)
        pend_A = pltpu.async_copy(
            bufA, out_hbm.at[b, pl.ds(d0, DG), pl.ds(0, HA)], sem_wA)

        if pend_B is not None:
            pend_B.wait()

        @plsc.parallel_loop(0, (WB - 1) // LANES, unroll=8)
        def _stepB(j):
            off = pl.multiple_of(j * LANES, LANES)
            iv = idxb[pl.ds(HA + off, LANES)]
            for dd in range(DG):
                gv = plsc.load_gather(tvm, [iv + (dd * TABLE)])
                bufB[dd, pl.ds(off, LANES)] = gv

        ivt = idxb[pl.ds(N, LANES)]
        for dd in range(DG):
            gv = plsc.load_gather(tvm, [ivt + (dd * TABLE)])
            plsc.store_scatter(bufB, [jnp.full((LANES,), dd, jnp.int32),
                                      jnp.full((LANES,), WB - 1, jnp.int32)],
                               gv, mask=iota < 1)
        pend_B = pltpu.async_copy(
            bufB, out_hbm.at[b, pl.ds(d0, DG), pl.ds(HA, WB)], sem_wB)

    pend_A.wait()
    pend_B.wait()


@functools.partial(
    pl.kernel,
    mesh=plsc.VectorSubcoreMesh(core_axis_name="c", subcore_axis_name="s"),
    compiler_params=pltpu.CompilerParams(use_tc_tiling_on_sc=True,
                                         needs_layout_passes=False),
    out_type=jax.ShapeDtypeStruct((B, D, N + 1), jnp.float32),
    scratch_types=[
        pltpu.VMEM_SHARED((NB_SC * SLOT,), jnp.int32),
        pltpu.VMEM((SLOT,), jnp.int32),
        pltpu.VMEM((CW,), jnp.float32),
        pltpu.VMEM((CW,), jnp.float32),
        pltpu.VMEM((CW,), jnp.float32),
        pltpu.VMEM((DG * TABLE,), jnp.float32),
        pltpu.VMEM((DG, HA), jnp.float32),
        pltpu.VMEM((DG, WB), jnp.float32),
        pltpu.VMEM((D + LANES,), jnp.float32),
        pltpu.SemaphoreType.DMA,
        pltpu.SemaphoreType.DMA,
        pltpu.SemaphoreType.DMA,
        pltpu.SemaphoreType.DMA,
    ],
)
def _sc_kernel(x_hbm, y_hbm, z_hbm, op_hbm, t_hbm, out_hbm,
               idx_all, idxb, xv, yv, zv, tvm, bufA, bufB, opvm,
               sem_tab, sem_wA, sem_wB, sem_pub):
    _sc_body(x_hbm, y_hbm, z_hbm, op_hbm, t_hbm, out_hbm,
             idx_all, idxb, xv, yv, zv, tvm, bufA, bufB, opvm,
             sem_tab, sem_wA, sem_wB, sem_pub)


def kernel(x, y, z, outputPos, posEmbeddingList):
    tflat = posEmbeddingList.T.reshape(-1)
    op_pad = jnp.pad(outputPos.reshape(-1), (0, LANES))
    out_t = _sc_kernel(x.reshape(-1), y.reshape(-1), z.reshape(-1),
                       op_pad, tflat)
    return out_t.transpose(0, 2, 1)
